# baseline (speedup 1.0000x reference)
import functools

import jax
import jax.numpy as jnp
from jax import lax
from jax.experimental import pallas as pl
from jax.experimental.pallas import tpu as pltpu
from jax.experimental.pallas import tpu_sc as plsc

_N = 50000
_E = 1600000
_NW = 32
_EPT = _E // _NW
_C = 2000
_NCH = _EPT // _C
_DT = 30.0
_NG = 16.0


def _sc_scatter_build():
    mesh = plsc.VectorSubcoreMesh(core_axis_name="c", subcore_axis_name="s")

    @functools.partial(
        pl.kernel,
        mesh=mesh,
        out_type=jax.ShapeDtypeStruct((_NW, _N), jnp.float32),
        compiler_params=pltpu.CompilerParams(needs_layout_passes=False),
        scratch_types=[
            pltpu.VMEM((_N,), jnp.float32),
            pltpu.VMEM((_C,), jnp.int32),
            pltpu.VMEM((_C,), jnp.int32),
            pltpu.VMEM((2 * _C,), jnp.float32),
            pltpu.VMEM((_C,), jnp.int32),
            pltpu.VMEM((_C,), jnp.int32),
            pltpu.VMEM((2 * _C,), jnp.float32),
            pltpu.VMEM((32,), jnp.float32),
            pltpu.SemaphoreType.DMA,
            pltpu.SemaphoreType.DMA,
        ],
    )
    def sc_scatter(ei_hbm, eraw_hbm, cst_hbm, out_hbm,
                   acc, row0, col0, flow0, row1, col1, flow1, cstv,
                   sem0, sem1):
        wid = lax.axis_index("s") * 2 + lax.axis_index("c")
        base = wid * _EPT
        pltpu.sync_copy(cst_hbm, cstv)
        scale = cstv[pl.ds(0, 16)]
        mean = cstv[pl.ds(16, 16)]
        zero = jnp.zeros((16,), jnp.float32)
        iota16 = lax.iota(jnp.int32, 16)
        bufs = ((row0, col0, flow0, sem0), (row1, col1, flow1, sem1))

        def zbody(i, carry):
            acc[pl.ds(i * 16, 16)] = zero
            return carry

        lax.fori_loop(0, _N // 16, zbody, 0, unroll=8)

        def start(ci, b):
            rv, cv, fv, sem = bufs[b]
            off = base + ci * _C
            pltpu.async_copy(ei_hbm.at[pl.ds(off, _C)], rv, sem)
            pltpu.async_copy(ei_hbm.at[pl.ds(_E + off, _C)], cv, sem)
            pltpu.async_copy(eraw_hbm.at[pl.ds(2 * off, 2 * _C)], fv, sem)

        def drain_and_scatter(b, carry):
            rv, cv, fv, sem = bufs[b]
            pltpu.make_async_copy(ei_hbm.at[pl.ds(0, _C)], rv, sem).wait()
            pltpu.make_async_copy(ei_hbm.at[pl.ds(0, _C)], cv, sem).wait()
            pltpu.make_async_copy(eraw_hbm.at[pl.ds(0, 2 * _C)], fv, sem).wait()

            def ebody(i, c2):
                sl = pl.ds(i * 16, 16)
                raw = plsc.load_gather(fv, [(i * 32) + iota16 * 2])
                f = raw * scale + mean
                plsc.addupdate_scatter(acc, [cv[sl]], f)
                plsc.addupdate_scatter(acc, [rv[sl]], -f)
                return c2

            return lax.fori_loop(0, _C // 16, ebody, carry, unroll=8)

        start(0, 0)

        def pair(pi, carry):
            c = 2 * pi
            start(c + 1, 1)
            carry = drain_and_scatter(0, carry)
            start(c + 2, 0)
            carry = drain_and_scatter(1, carry)
            return carry

        carry = lax.fori_loop(0, (_NCH - 1) // 2, pair, 0)
        drain_and_scatter(0, carry)
        pltpu.sync_copy(acc, out_hbm.at[wid])

    return sc_scatter


@functools.cache
def _sc_scatter():
    return _sc_scatter_build()


def _fin_body(part_ref, p0_ref, i0_ref, rain_ref, mask_ref, std_ref, out_ref):
    net = jnp.sum(part_ref[...], axis=0, keepdims=True)
    d = (p0_ref[...] - i0_ref[...]) * std_ref[...][0, 0]
    err = d - _DT * net - rain_ref[...]
    tot = jnp.sum(mask_ref[...] * jnp.abs(err))
    out_ref[...] = (tot / _NG).reshape(1, 1)


def _finalize(partials, p0, i0, rain, maskf, std):
    return pl.pallas_call(
        _fin_body,
        out_shape=jax.ShapeDtypeStruct((1, 1), jnp.float32),
    )(partials, p0, i0, rain, maskf, std)


def kernel(batch_node_pred, batch_node_input, batch_edge_input, rainfall,
           node_mean, node_std, edge_mean, edge_std,
           edge_index, batch, node_filter_mask):
    cst = jnp.concatenate([
        jnp.broadcast_to(edge_std[0], (16,)),
        jnp.broadcast_to(edge_mean[0], (16,)),
    ]).astype(jnp.float32)
    partials = _sc_scatter()(edge_index.reshape(-1), batch_edge_input.reshape(-1), cst)

    p0 = batch_node_pred[:, 0].reshape(1, _N)
    i0 = batch_node_input[:, 0].reshape(1, _N)
    rain = rainfall.reshape(1, _N)
    maskf = node_filter_mask.astype(jnp.float32).reshape(1, _N)
    std = node_std[0].reshape(1, 1)
    loss = _finalize(partials, p0, i0, rain, maskf, std)
    return loss[0, 0]


# speedup vs baseline: 29.4826x; 29.4826x over previous
import functools

import jax
import jax.numpy as jnp
from jax import lax
from jax.experimental import pallas as pl
from jax.experimental.pallas import tpu as pltpu
from jax.experimental.pallas import tpu_sc as plsc

_N = 50000
_E = 1600000
_NW = 32
_NBT = _E // 128
_BPC = 16
_CE = _BPC * 128
_MAIN = 24
_TAIL = 6
_XTRA = _NBT - 390 * _NW
_DT = 30.0
_NG = 16.0


def _sc_scatter_build():
    mesh = plsc.VectorSubcoreMesh(core_axis_name="c", subcore_axis_name="s")

    @functools.partial(
        pl.kernel,
        mesh=mesh,
        out_type=jax.ShapeDtypeStruct((_NW, _N), jnp.float32),
        compiler_params=pltpu.CompilerParams(needs_layout_passes=False),
        scratch_types=[
            pltpu.VMEM((_N,), jnp.float32),
            pltpu.VMEM((2, _CE), jnp.int32),
            pltpu.VMEM((2, _CE), jnp.float32),
            pltpu.VMEM((2, _CE), jnp.int32),
            pltpu.VMEM((2, _CE), jnp.float32),
            pltpu.VMEM((32,), jnp.float32),
            pltpu.SemaphoreType.DMA,
            pltpu.SemaphoreType.DMA,
        ],
    )
    def sc_scatter(ei_hbm, ef_hbm, cst_hbm, out_hbm,
                   acc, iv0, fv0, iv1, fv1, cstv, sem0, sem1):
        wid = lax.axis_index("s") * 2 + lax.axis_index("c")
        start_burst = 390 * wid + jnp.minimum(wid, _XTRA)
        pltpu.sync_copy(cst_hbm, cstv)
        scale = cstv[pl.ds(0, 16)]
        mean = cstv[pl.ds(16, 16)]
        zero = jnp.zeros((16,), jnp.float32)

        def zbody(i, carry):
            acc[pl.ds(i * 16, 16)] = zero
            return carry

        lax.fori_loop(0, _N // 16, zbody, 0, unroll=8)

        def start_main(ci, iv, fv, sem):
            off = (start_burst + ci * _BPC) * 128
            pltpu.async_copy(ei_hbm.at[:, pl.ds(off, _CE)], iv, sem)
            pltpu.async_copy(ef_hbm.at[:, pl.ds(off, _CE)], fv, sem)

        def drain(iv, fv, sem):
            pltpu.make_async_copy(ei_hbm.at[:, pl.ds(0, _CE)], iv, sem).wait()
            pltpu.make_async_copy(ef_hbm.at[:, pl.ds(0, _CE)], fv, sem).wait()

        def scat(iv, fv, iters, carry):
            def ebody(i, c2):
                sl = pl.ds(i * 16, 16)
                f = fv[0, sl] * scale + mean
                plsc.addupdate_scatter(acc, [iv[1, sl]], f)
                plsc.addupdate_scatter(acc, [iv[0, sl]], -f)
                return c2

            return lax.fori_loop(0, iters, ebody, carry, unroll=8)

        start_main(0, iv0, fv0, sem0)

        def pair(pi, carry):
            start_main(2 * pi + 1, iv1, fv1, sem1)
            drain(iv0, fv0, sem0)
            carry = scat(iv0, fv0, _CE // 16, carry)

            @pl.when(2 * pi + 2 < _MAIN)
            def _():
                start_main(2 * pi + 2, iv0, fv0, sem0)

            drain(iv1, fv1, sem1)
            carry = scat(iv1, fv1, _CE // 16, carry)
            return carry

        carry = lax.fori_loop(0, _MAIN // 2, pair, 0)

        off_a = (start_burst + _MAIN * _BPC) * 128
        na = _TAIL * 128
        pltpu.sync_copy(ei_hbm.at[:, pl.ds(off_a, na)], iv0.at[:, pl.ds(0, na)])
        pltpu.sync_copy(ef_hbm.at[:, pl.ds(off_a, na)], fv0.at[:, pl.ds(0, na)])
        carry = scat(iv0, fv0, na // 16, carry)

        @pl.when(wid < _XTRA)
        def _():
            off_b = (start_burst + _MAIN * _BPC + _TAIL) * 128
            pltpu.sync_copy(ei_hbm.at[:, pl.ds(off_b, 128)], iv1.at[:, pl.ds(0, 128)])
            pltpu.sync_copy(ef_hbm.at[:, pl.ds(off_b, 128)], fv1.at[:, pl.ds(0, 128)])
            scat(iv1, fv1, 128 // 16, 0)

        pltpu.sync_copy(acc, out_hbm.at[wid])

    return sc_scatter


@functools.cache
def _sc_scatter():
    return _sc_scatter_build()


def _fin_body(part_ref, pred_ref, in_ref, rain_ref, mask_ref, std_ref, out_ref):
    net = jnp.sum(part_ref[...], axis=0)
    d = (pred_ref[0] - in_ref[0]) * std_ref[0]
    err = d - _DT * net - rain_ref[...]
    tot = jnp.sum(mask_ref[...] * jnp.abs(err))
    out_ref[...] = (tot / _NG).reshape(1, 1)


def _finalize(partials, pred_t, in_t, rain, maskf, std):
    return pl.pallas_call(
        _fin_body,
        out_shape=jax.ShapeDtypeStruct((1, 1), jnp.float32),
    )(partials, pred_t, in_t, rain, maskf, std)


def kernel(batch_node_pred, batch_node_input, batch_edge_input, rainfall,
           node_mean, node_std, edge_mean, edge_std,
           edge_index, batch, node_filter_mask):
    cst = jnp.concatenate([
        jnp.broadcast_to(edge_std[0], (16,)),
        jnp.broadcast_to(edge_mean[0], (16,)),
    ]).astype(jnp.float32)
    ef = lax.transpose(batch_edge_input, (1, 0))
    partials = _sc_scatter()(edge_index, ef, cst)

    pred_t = lax.transpose(batch_node_pred, (1, 0))
    in_t = lax.transpose(batch_node_input, (1, 0))
    maskf = node_filter_mask.astype(jnp.float32)
    std = node_std[0].reshape(1)
    loss = _finalize(partials, pred_t, in_t, rainfall, maskf, std)
    return loss[0, 0]


# speedup vs baseline: 34.8309x; 1.1814x over previous
import functools

import jax
import jax.numpy as jnp
from jax import lax
from jax.experimental import pallas as pl
from jax.experimental.pallas import tpu as pltpu
from jax.experimental.pallas import tpu_sc as plsc

_N = 50000
_E = 1600000
_NW = 32
_NBT = _E // 128
_BPC = 16
_CE = _BPC * 128
_MAIN = 24
_TAIL = 6
_XTRA = _NBT - 390 * _NW
_DT = 30.0
_NG = 16.0


def _sc_scatter_build():
    mesh = plsc.VectorSubcoreMesh(core_axis_name="c", subcore_axis_name="s")

    @functools.partial(
        pl.kernel,
        mesh=mesh,
        out_type=jax.ShapeDtypeStruct((_NW, _N), jnp.float32),
        compiler_params=pltpu.CompilerParams(needs_layout_passes=False),
        scratch_types=[
            pltpu.VMEM((_N,), jnp.float32),
            pltpu.VMEM((_N,), jnp.float32),
            pltpu.VMEM((2, _CE), jnp.int32),
            pltpu.VMEM((2, _CE), jnp.float32),
            pltpu.VMEM((2, _CE), jnp.int32),
            pltpu.VMEM((2, _CE), jnp.float32),
            pltpu.VMEM((32,), jnp.float32),
            pltpu.SemaphoreType.DMA,
            pltpu.SemaphoreType.DMA,
        ],
    )
    def sc_scatter(ei_hbm, ef_hbm, cst_hbm, out_hbm,
                   acc_p, acc_m, iv0, fv0, iv1, fv1, cstv, sem0, sem1):
        wid = lax.axis_index("s") * 2 + lax.axis_index("c")
        start_burst = 390 * wid + jnp.minimum(wid, _XTRA)
        pltpu.sync_copy(cst_hbm, cstv)
        scale = cstv[pl.ds(0, 16)]
        mean = cstv[pl.ds(16, 16)]
        zero = jnp.zeros((16,), jnp.float32)

        def zbody(i, carry):
            acc_p[pl.ds(i * 16, 16)] = zero
            acc_m[pl.ds(i * 16, 16)] = zero
            return carry

        lax.fori_loop(0, _N // 16, zbody, 0, unroll=8)

        def start_main(ci, iv, fv, sem):
            off = (start_burst + ci * _BPC) * 128
            pltpu.async_copy(ei_hbm.at[:, pl.ds(off, _CE)], iv, sem)
            pltpu.async_copy(ef_hbm.at[:, pl.ds(off, _CE)], fv, sem)

        def drain(iv, fv, sem):
            pltpu.make_async_copy(ei_hbm.at[:, pl.ds(0, _CE)], iv, sem).wait()
            pltpu.make_async_copy(ef_hbm.at[:, pl.ds(0, _CE)], fv, sem).wait()

        def scat(iv, fv, iters, carry):
            @functools.partial(plsc.parallel_loop, 0, iters, unroll=8)
            def _(i):
                sl = pl.ds(i * 16, 16)
                f = fv[0, sl] * scale + mean
                plsc.addupdate_scatter(acc_p, [iv[1, sl]], f)
                plsc.addupdate_scatter(acc_m, [iv[0, sl]], f)

            return carry

        start_main(0, iv0, fv0, sem0)

        def pair(pi, carry):
            start_main(2 * pi + 1, iv1, fv1, sem1)
            drain(iv0, fv0, sem0)
            carry = scat(iv0, fv0, _CE // 16, carry)

            @pl.when(2 * pi + 2 < _MAIN)
            def _():
                start_main(2 * pi + 2, iv0, fv0, sem0)

            drain(iv1, fv1, sem1)
            carry = scat(iv1, fv1, _CE // 16, carry)
            return carry

        carry = lax.fori_loop(0, _MAIN // 2, pair, 0)

        off_a = (start_burst + _MAIN * _BPC) * 128
        na = _TAIL * 128
        pltpu.sync_copy(ei_hbm.at[:, pl.ds(off_a, na)], iv0.at[:, pl.ds(0, na)])
        pltpu.sync_copy(ef_hbm.at[:, pl.ds(off_a, na)], fv0.at[:, pl.ds(0, na)])
        carry = scat(iv0, fv0, na // 16, carry)

        @pl.when(wid < _XTRA)
        def _():
            off_b = (start_burst + _MAIN * _BPC + _TAIL) * 128
            pltpu.sync_copy(ei_hbm.at[:, pl.ds(off_b, 128)], iv1.at[:, pl.ds(0, 128)])
            pltpu.sync_copy(ef_hbm.at[:, pl.ds(off_b, 128)], fv1.at[:, pl.ds(0, 128)])
            scat(iv1, fv1, 128 // 16, 0)

        def mbody(i, carry):
            sl = pl.ds(i * 16, 16)
            acc_p[sl] = acc_p[sl] - acc_m[sl]
            return carry

        lax.fori_loop(0, _N // 16, mbody, 0, unroll=8)
        pltpu.sync_copy(acc_p, out_hbm.at[wid])

    return sc_scatter


@functools.cache
def _sc_scatter():
    return _sc_scatter_build()


def _fin_body(part_ref, pred_ref, in_ref, rain_ref, mask_ref, std_ref, out_ref):
    net = jnp.sum(part_ref[...], axis=0)
    d = (pred_ref[0] - in_ref[0]) * std_ref[0]
    err = d - _DT * net - rain_ref[...]
    tot = jnp.sum(mask_ref[...] * jnp.abs(err))
    out_ref[...] = (tot / _NG).reshape(1, 1)


def _finalize(partials, pred_t, in_t, rain, maskf, std):
    return pl.pallas_call(
        _fin_body,
        out_shape=jax.ShapeDtypeStruct((1, 1), jnp.float32),
    )(partials, pred_t, in_t, rain, maskf, std)


def kernel(batch_node_pred, batch_node_input, batch_edge_input, rainfall,
           node_mean, node_std, edge_mean, edge_std,
           edge_index, batch, node_filter_mask):
    cst = jnp.concatenate([
        jnp.broadcast_to(edge_std[0], (16,)),
        jnp.broadcast_to(edge_mean[0], (16,)),
    ]).astype(jnp.float32)
    ef = lax.transpose(batch_edge_input, (1, 0))
    partials = _sc_scatter()(edge_index, ef, cst)

    pred_t = lax.transpose(batch_node_pred, (1, 0))
    in_t = lax.transpose(batch_node_input, (1, 0))
    maskf = node_filter_mask.astype(jnp.float32)
    std = node_std[0].reshape(1)
    loss = _finalize(partials, pred_t, in_t, rainfall, maskf, std)
    return loss[0, 0]
